# baseline (speedup 1.0000x reference)
import functools
import math

import jax
import jax.numpy as jnp
import numpy as np
from jax import lax
from jax.experimental import pallas as pl
from jax.experimental.pallas import tpu as pltpu
from jax.experimental.pallas import tpu_sc as plsc

GRID = 128
FEAT = 64
HID = 64
NRAYS = 8192
DELTA_MIN = math.sqrt(3.0) / 1024.0


def _steps():
    t = 0.0
    ts = [t]
    ds = []
    while t < 100000.0:
        s = min(1e10, max(DELTA_MIN, t))
        t += s
        ts.append(t)
        ds.append(s)
    return np.asarray(ts[:-1], np.float32), np.asarray(ds, np.float32)


_T, _D = _steps()
NS = _T.shape[0]
NSP = 32
NROWS = NRAYS * NSP
RB = 256
ROWS_B = RB * NSP
NBLK = NRAYS // RB
NWORDS = GRID * GRID * (GRID // 32)
PACK_ROWS = 8192
NW = 32
PER_W = NROWS // NW
VECS = PER_W // 16

_T_PAD = np.concatenate([_T, np.zeros(NSP - NS, np.float32)])
_D_PAD = np.concatenate([_D, np.zeros(NSP - NS, np.float32)])
_T_COL = np.tile(_T_PAD, RB)[:, None]
_D_COL = np.tile(_D_PAD, RB)[:, None]
_S_COL = np.tile(np.arange(NSP, dtype=np.int32), RB)[:, None]


def _pack_body(g_ref, out_ref):
    b = (g_ref[...] != 0.0).astype(jnp.int32)
    sh = lax.broadcasted_iota(jnp.int32, b.shape, 1)
    out_ref[...] = jnp.sum(b << sh, axis=1, keepdims=True)


def _expand_rays(r_ref):
    return r_ref[...]


def _contracted(o3, d3, t):
    pos = o3 + d3 * t
    nrm = jnp.max(jnp.abs(pos), axis=-1, keepdims=True)
    safe = jnp.maximum(nrm, 1e-12)
    return jnp.where(nrm <= 1.0, pos, (2.0 - 1.0 / safe) * pos / safe) / 2.0


def _codes_body(o_ref, d_ref, t_ref, sidx_ref, out_ref):
    o3 = _expand_rays(o_ref)
    d3 = _expand_rays(d_ref)
    flat = _contracted(o3, d3, t_ref[...])
    code = jnp.zeros((ROWS_B, 1), jnp.int32)
    for axis, bit in ((0, 0), (1, 8), (2, 16)):
        iv = ((flat[:, axis:axis + 1] + 1.0) * GRID - 1.0) / 2.0
        vf = jnp.floor(iv)
        vi = vf.astype(jnp.int32) + 1
        frac = (iv > vf).astype(jnp.int32)
        code = code + vi * jnp.int32(1 << bit) + frac * jnp.int32(1 << (24 + bit // 8))
    code = jnp.where(sidx_ref[...] < NS, code, jnp.int32(0x00FFFFFF))
    out_ref[...] = code


def _main_body(o_ref, d_ref, mask_ref, t_ref, dist_ref, sidx_ref,
               W1_ref, b1_ref, W2_ref, b2_ref, Ws_ref, bs_ref,
               Wr1a_ref, Wr1b_ref, br1_ref, Wr2_ref, br2_ref, out_ref):
    o3 = _expand_rays(o_ref)
    d3 = _expand_rays(d_ref)
    flat = _contracted(o3, d3, t_ref[...])
    mrow = mask_ref[...]
    h = jnp.maximum(jnp.dot(flat, W1_ref[...]) + b1_ref[...], 0.0)
    feat = (jnp.dot(h, W2_ref[...]) + b2_ref[...]) * mrow
    sig = jnp.dot(feat, Ws_ref[...]) + bs_ref[...]
    sigma = jax.nn.softplus(sig) * mrow
    alpha = -sigma * dist_ref[...]
    sidx = sidx_ref[...]
    c = alpha
    for k in (1, 2, 4, 8, 16):
        c = c + jnp.where(sidx >= k, pltpu.roll(c, k, 0), 0.0)
    trans = jnp.exp(c - alpha)
    a = 1.0 - jnp.exp(alpha)
    w = trans * a
    m2 = jnp.where((mrow > 0.0) & (w > 1e-4), 1.0, 0.0)
    hr = jnp.maximum(jnp.dot(feat, Wr1a_ref[...])
                     + jnp.dot(d3, Wr1b_ref[...]) + br1_ref[...], 0.0)
    rgb = jax.nn.sigmoid(jnp.dot(hr, Wr2_ref[...]) + br2_ref[...])
    contrib = rgb * (w * m2)
    out_ref[...] = jnp.sum(contrib.reshape(RB, NSP, 3), axis=1)


def _sc_mask_body(words_hbm, codes_hbm, out_hbm, bits_v, codes_v, mask_v):
    nc = 2
    wid = lax.axis_index("s") * nc + lax.axis_index("c")
    base = wid * PER_W
    pltpu.sync_copy(words_hbm, bits_v)
    pltpu.sync_copy(codes_hbm.at[pl.ds(base, PER_W)], codes_v)

    def body(i, carry):
        code = codes_v[pl.ds(i * 16, 16)]
        x0 = (code & 255) - 1
        y0 = ((code >> 8) & 255) - 1
        z0 = ((code >> 16) & 255) - 1
        fx = ((code >> 24) & 1) == 1
        fy = ((code >> 25) & 1) == 1
        fz = ((code >> 26) & 1) == 1
        lim = GRID - 1
        okx = ((x0 >= 0) & (x0 <= lim), fx & (x0 + 1 <= lim) & (x0 + 1 >= 0))
        oky = ((y0 >= 0) & (y0 <= lim), fy & (y0 + 1 <= lim) & (y0 + 1 >= 0))
        okz = ((z0 >= 0) & (z0 <= lim), fz & (z0 + 1 <= lim) & (z0 + 1 >= 0))
        m = None
        for dz in (0, 1):
            for dy in (0, 1):
                yz = (z0 + dz) * GRID + (y0 + dy)
                for dx in (0, 1):
                    x = x0 + dx
                    wi = (yz * 4 + (x >> 5)) & (NWORDS - 1)
                    wv = plsc.load_gather(bits_v, [wi])
                    hit = (((wv >> (x & 31)) & 1) == 1)
                    hit = hit & okz[dz] & oky[dy] & okx[dx]
                    m = hit if m is None else (m | hit)
        mask_v[pl.ds(i * 16, 16)] = jnp.where(m, 1.0, 0.0)
        return carry

    lax.fori_loop(0, VECS, body, 0)
    pltpu.sync_copy(mask_v, out_hbm.at[pl.ds(base, PER_W)])


@functools.cache
def _sc_mask_call():
    mesh = plsc.VectorSubcoreMesh(core_axis_name="c", subcore_axis_name="s")
    return pl.kernel(
        _sc_mask_body,
        out_type=jax.ShapeDtypeStruct((NROWS,), jnp.float32),
        mesh=mesh,
        compiler_params=pltpu.CompilerParams(needs_layout_passes=False),
        scratch_types=[
            pltpu.VMEM((NWORDS,), jnp.int32),
            pltpu.VMEM((PER_W,), jnp.int32),
            pltpu.VMEM((PER_W,), jnp.float32),
        ],
    )


@functools.cache
def _pack_call():
    return pl.pallas_call(
        _pack_body,
        grid=(NWORDS // PACK_ROWS,),
        in_specs=[pl.BlockSpec((PACK_ROWS, 32), lambda i: (i, 0))],
        out_specs=pl.BlockSpec((PACK_ROWS, 1), lambda i: (i, 0)),
        out_shape=jax.ShapeDtypeStruct((NWORDS, 1), jnp.int32),
    )


@functools.cache
def _codes_call():
    return pl.pallas_call(
        _codes_body,
        grid=(NBLK,),
        in_specs=[
            pl.BlockSpec((ROWS_B, 3), lambda i: (i, 0)),
            pl.BlockSpec((ROWS_B, 3), lambda i: (i, 0)),
            pl.BlockSpec((ROWS_B, 1), lambda i: (0, 0)),
            pl.BlockSpec((ROWS_B, 1), lambda i: (0, 0)),
        ],
        out_specs=pl.BlockSpec((ROWS_B, 1), lambda i: (i, 0)),
        out_shape=jax.ShapeDtypeStruct((NROWS, 1), jnp.int32),
    )


@functools.cache
def _main_call():
    full = lambda i: (0, 0)
    return pl.pallas_call(
        _main_body,
        grid=(NBLK,),
        in_specs=[
            pl.BlockSpec((ROWS_B, 3), lambda i: (i, 0)),
            pl.BlockSpec((ROWS_B, 3), lambda i: (i, 0)),
            pl.BlockSpec((ROWS_B, 1), lambda i: (i, 0)),
            pl.BlockSpec((ROWS_B, 1), full),
            pl.BlockSpec((ROWS_B, 1), full),
            pl.BlockSpec((ROWS_B, 1), full),
            pl.BlockSpec((3, HID), full),
            pl.BlockSpec((1, HID), full),
            pl.BlockSpec((HID, FEAT), full),
            pl.BlockSpec((1, FEAT), full),
            pl.BlockSpec((FEAT, 1), full),
            pl.BlockSpec((1, 1), full),
            pl.BlockSpec((FEAT, HID), full),
            pl.BlockSpec((3, HID), full),
            pl.BlockSpec((1, HID), full),
            pl.BlockSpec((HID, 3), full),
            pl.BlockSpec((1, 3), full),
        ],
        out_specs=pl.BlockSpec((RB, 3), lambda i: (i, 0)),
        out_shape=jax.ShapeDtypeStruct((NRAYS, 3), jnp.float32),
    )


def kernel(rays_o, rays_d, grid, W1, b1, W2, b2, Ws, bs, Wr1, br1, Wr2, br2):
    tcol = jnp.asarray(_T_COL)
    dcol = jnp.asarray(_D_COL)
    scol = jnp.asarray(_S_COL)
    o_rows = jnp.repeat(rays_o, NSP, axis=0)
    d_rows = jnp.repeat(rays_d, NSP, axis=0)
    words = _pack_call()(grid.reshape(NWORDS, 32)).reshape(NWORDS)
    codes = _codes_call()(o_rows, d_rows, tcol, scol).reshape(NROWS)
    maskf = _sc_mask_call()(words, codes)
    out = _main_call()(
        o_rows, d_rows, maskf.reshape(NROWS, 1), tcol, dcol, scol,
        W1, b1.reshape(1, HID), W2, b2.reshape(1, FEAT),
        Ws, bs.reshape(1, 1),
        Wr1[:FEAT], Wr1[FEAT:], br1.reshape(1, HID),
        Wr2, br2.reshape(1, 3))
    return out


# speedup vs baseline: 3.8768x; 3.8768x over previous
import functools
import math

import jax
import jax.numpy as jnp
import numpy as np
from jax import lax
from jax.experimental import pallas as pl
from jax.experimental.pallas import tpu as pltpu
from jax.experimental.pallas import tpu_sc as plsc

GRID = 128
FEAT = 64
HID = 64
NRAYS = 8192
DELTA_MIN = math.sqrt(3.0) / 1024.0


def _steps():
    t = 0.0
    ts = [t]
    ds = []
    while t < 100000.0:
        s = min(1e10, max(DELTA_MIN, t))
        t += s
        ts.append(t)
        ds.append(s)
    return np.asarray(ts[:-1], np.float32), np.asarray(ds, np.float32)


_T, _D = _steps()
NS = _T.shape[0]
NSP = 32
NROWS = NRAYS * NSP
RB = 256
CB = RB * NSP
NBLK = NRAYS // RB
NWORDS = GRID * GRID * (GRID // 32)
PACK_ROWS = 8192
NW = 32
PER_W = NROWS // NW
VECS = PER_W // 16

_T_PAD = np.concatenate([_T, np.zeros(NSP - NS, np.float32)])
_D_PAD = np.concatenate([_D, np.zeros(NSP - NS, np.float32)])
_T_ROW = np.tile(_T_PAD, RB)[None, :]
_D_ROW = np.tile(_D_PAD, RB)[None, :]
_S_ROW = np.tile(np.arange(NSP, dtype=np.int32), RB)[None, :]
_SEL = np.zeros((CB, RB), np.float32)
_SEL[np.arange(CB), np.arange(CB) // NSP] = 1.0


def _pack_body(g_ref, out_ref):
    b = (g_ref[...] != 0.0).astype(jnp.int32)
    sh = lax.broadcasted_iota(jnp.int32, b.shape, 1)
    out_ref[...] = jnp.sum(b << sh, axis=1, keepdims=True)


def _contracted(o3, d3, t):
    pos = o3 + d3 * t
    nrm = jnp.max(jnp.abs(pos), axis=0, keepdims=True)
    safe = jnp.maximum(nrm, 1e-12)
    return jnp.where(nrm <= 1.0, pos, (2.0 - 1.0 / safe) * pos / safe) / 2.0


def _codes_body(o_ref, d_ref, t_ref, sidx_ref, out_ref):
    flat = _contracted(o_ref[...], d_ref[...], t_ref[...])
    iv3 = ((flat + 1.0) * GRID - 1.0) / 2.0
    vf3 = jnp.floor(iv3)
    vi3 = vf3.astype(jnp.int32) + 1
    fr3 = (iv3 > vf3).astype(jnp.int32)
    code = jnp.zeros((1, CB), jnp.int32)
    for axis in range(3):
        code = code + (vi3[axis:axis + 1, :] * jnp.int32(1 << (8 * axis))
                       + fr3[axis:axis + 1, :] * jnp.int32(1 << (24 + axis)))
    code = jnp.where(sidx_ref[...] < NS, code, jnp.int32(0x00FFFFFF))
    out_ref[...] = code


def _main_body(o_ref, d_ref, mask_ref, t_ref, dist_ref, sidx_ref, sel_ref,
               W1_ref, b1_ref, W2_ref, b2_ref, Ws_ref, bs_ref,
               Wr1a_ref, Wr1b_ref, br1_ref, Wr2_ref, br2_ref, out_ref):
    d3 = d_ref[...]
    flat = _contracted(o_ref[...], d3, t_ref[...])
    msk = mask_ref[...]
    h = jnp.maximum(jnp.dot(W1_ref[...], flat) + b1_ref[...], 0.0)
    feat = (jnp.dot(W2_ref[...], h) + b2_ref[...]) * msk
    sig = jnp.dot(Ws_ref[...], feat) + bs_ref[...]
    sigma = jax.nn.softplus(sig) * msk
    alpha = -sigma * dist_ref[...]
    sidx = sidx_ref[...]
    c = alpha
    for k in (1, 2, 4, 8, 16):
        c = c + jnp.where(sidx >= k, pltpu.roll(c, k, 1), 0.0)
    trans = jnp.exp(c - alpha)
    a = 1.0 - jnp.exp(alpha)
    w = trans * a
    wm2 = jnp.where((msk > 0.0) & (w > 1e-4), w, 0.0)
    hr = jnp.maximum(jnp.dot(Wr1a_ref[...], feat)
                     + jnp.dot(Wr1b_ref[...], d3) + br1_ref[...], 0.0)
    rgb = jax.nn.sigmoid(jnp.dot(Wr2_ref[...], hr) + br2_ref[...])
    contrib = rgb * wm2
    out_ref[...] = jnp.dot(contrib, sel_ref[...])


def _sc_mask_body(words_hbm, codes_hbm, out_hbm, bits_v, codes_v, mask_v):
    nc = 2
    wid = lax.axis_index("s") * nc + lax.axis_index("c")
    base = wid * PER_W
    pltpu.sync_copy(words_hbm, bits_v)
    pltpu.sync_copy(codes_hbm.at[pl.ds(base, PER_W)], codes_v)

    def body(i, carry):
        code = codes_v[pl.ds(i * 16, 16)]
        x0 = (code & 255) - 1
        y0 = ((code >> 8) & 255) - 1
        z0 = ((code >> 16) & 255) - 1
        fx = ((code >> 24) & 1) == 1
        fy = ((code >> 25) & 1) == 1
        fz = ((code >> 26) & 1) == 1
        lim = GRID - 1
        okx = ((x0 >= 0) & (x0 <= lim), fx & (x0 + 1 <= lim) & (x0 + 1 >= 0))
        oky = ((y0 >= 0) & (y0 <= lim), fy & (y0 + 1 <= lim) & (y0 + 1 >= 0))
        okz = ((z0 >= 0) & (z0 <= lim), fz & (z0 + 1 <= lim) & (z0 + 1 >= 0))
        m = None
        for dz in (0, 1):
            for dy in (0, 1):
                yz = (z0 + dz) * GRID + (y0 + dy)
                for dx in (0, 1):
                    x = x0 + dx
                    wi = (yz * 4 + (x >> 5)) & (NWORDS - 1)
                    wv = plsc.load_gather(bits_v, [wi])
                    hit = (((wv >> (x & 31)) & 1) == 1)
                    hit = hit & okz[dz] & oky[dy] & okx[dx]
                    m = hit if m is None else (m | hit)
        mask_v[pl.ds(i * 16, 16)] = jnp.where(m, 1.0, 0.0)
        return carry

    lax.fori_loop(0, VECS, body, 0)
    pltpu.sync_copy(mask_v, out_hbm.at[pl.ds(base, PER_W)])


@functools.cache
def _sc_mask_call():
    mesh = plsc.VectorSubcoreMesh(core_axis_name="c", subcore_axis_name="s")
    return pl.kernel(
        _sc_mask_body,
        out_type=jax.ShapeDtypeStruct((NROWS,), jnp.float32),
        mesh=mesh,
        compiler_params=pltpu.CompilerParams(needs_layout_passes=False),
        scratch_types=[
            pltpu.VMEM((NWORDS,), jnp.int32),
            pltpu.VMEM((PER_W,), jnp.int32),
            pltpu.VMEM((PER_W,), jnp.float32),
        ],
    )


@functools.cache
def _pack_call():
    return pl.pallas_call(
        _pack_body,
        grid=(NWORDS // PACK_ROWS,),
        in_specs=[pl.BlockSpec((PACK_ROWS, 32), lambda i: (i, 0))],
        out_specs=pl.BlockSpec((PACK_ROWS, 1), lambda i: (i, 0)),
        out_shape=jax.ShapeDtypeStruct((NWORDS, 1), jnp.int32),
    )


@functools.cache
def _codes_call():
    full = lambda i: (0, 0)
    return pl.pallas_call(
        _codes_body,
        grid=(NBLK,),
        in_specs=[
            pl.BlockSpec((3, CB), lambda i: (0, i)),
            pl.BlockSpec((3, CB), lambda i: (0, i)),
            pl.BlockSpec((1, CB), full),
            pl.BlockSpec((1, CB), full),
        ],
        out_specs=pl.BlockSpec((1, CB), lambda i: (0, i)),
        out_shape=jax.ShapeDtypeStruct((1, NROWS), jnp.int32),
    )


@functools.cache
def _main_call():
    full = lambda i: (0, 0)
    return pl.pallas_call(
        _main_body,
        grid=(NBLK,),
        in_specs=[
            pl.BlockSpec((3, CB), lambda i: (0, i)),
            pl.BlockSpec((3, CB), lambda i: (0, i)),
            pl.BlockSpec((1, CB), lambda i: (0, i)),
            pl.BlockSpec((1, CB), full),
            pl.BlockSpec((1, CB), full),
            pl.BlockSpec((1, CB), full),
            pl.BlockSpec((CB, RB), full),
            pl.BlockSpec((HID, 3), full),
            pl.BlockSpec((HID, 1), full),
            pl.BlockSpec((FEAT, HID), full),
            pl.BlockSpec((FEAT, 1), full),
            pl.BlockSpec((1, FEAT), full),
            pl.BlockSpec((1, 1), full),
            pl.BlockSpec((HID, FEAT), full),
            pl.BlockSpec((HID, 3), full),
            pl.BlockSpec((HID, 1), full),
            pl.BlockSpec((3, HID), full),
            pl.BlockSpec((3, 1), full),
        ],
        out_specs=pl.BlockSpec((3, RB), lambda i: (0, i)),
        out_shape=jax.ShapeDtypeStruct((3, NRAYS), jnp.float32),
    )


def kernel(rays_o, rays_d, grid, W1, b1, W2, b2, Ws, bs, Wr1, br1, Wr2, br2):
    trow = jnp.asarray(_T_ROW)
    drow = jnp.asarray(_D_ROW)
    srow = jnp.asarray(_S_ROW)
    sel = jnp.asarray(_SEL)
    oT = jnp.repeat(rays_o.T, NSP, axis=1)
    dT = jnp.repeat(rays_d.T, NSP, axis=1)
    words = _pack_call()(grid.reshape(NWORDS, 32)).reshape(NWORDS)
    codes = _codes_call()(oT, dT, trow, srow).reshape(NROWS)
    maskf = _sc_mask_call()(words, codes)
    outT = _main_call()(
        oT, dT, maskf.reshape(1, NROWS), trow, drow, srow, sel,
        W1.T, b1.reshape(HID, 1), W2.T, b2.reshape(FEAT, 1),
        Ws.T, bs.reshape(1, 1),
        Wr1[:FEAT].T, Wr1[FEAT:].T, br1.reshape(HID, 1),
        Wr2.T, br2.reshape(3, 1))
    return outT.T


# speedup vs baseline: 3.8788x; 1.0005x over previous
import functools
import math

import jax
import jax.numpy as jnp
import numpy as np
from jax import lax
from jax.experimental import pallas as pl
from jax.experimental.pallas import tpu as pltpu
from jax.experimental.pallas import tpu_sc as plsc

GRID = 128
FEAT = 64
HID = 64
NRAYS = 8192
DELTA_MIN = math.sqrt(3.0) / 1024.0


def _steps():
    t = 0.0
    ts = [t]
    ds = []
    while t < 100000.0:
        s = min(1e10, max(DELTA_MIN, t))
        t += s
        ts.append(t)
        ds.append(s)
    return np.asarray(ts[:-1], np.float32), np.asarray(ds, np.float32)


_T, _D = _steps()
NS = _T.shape[0]
NSP = 32
NROWS = NRAYS * NSP
RB = 256
CB = RB * NSP
NBLK = NRAYS // RB
NWORDS = GRID * GRID * (GRID // 32)
PACK_ROWS = 8192
NW = 32
PER_W = NROWS // NW
VECS = PER_W // 16

_T_PAD = np.concatenate([_T, np.zeros(NSP - NS, np.float32)])
_D_PAD = np.concatenate([_D, np.zeros(NSP - NS, np.float32)])
_T_ROW = np.tile(_T_PAD, RB)[None, :]
_D_ROW = np.tile(_D_PAD, RB)[None, :]
_S_ROW = np.tile(np.arange(NSP, dtype=np.int32), RB)[None, :]
_SEL = np.zeros((CB, RB), np.float32)
_SEL[np.arange(CB), np.arange(CB) // NSP] = 1.0


def _pack_body(g_ref, out_ref):
    b = (g_ref[...] != 0.0).astype(jnp.int32)
    sh = lax.broadcasted_iota(jnp.int32, b.shape, 1)
    out_ref[...] = jnp.sum(b << sh, axis=1, keepdims=True)


def _contracted(o3, d3, t):
    pos = o3 + d3 * t
    nrm = jnp.max(jnp.abs(pos), axis=0, keepdims=True)
    safe = jnp.maximum(nrm, 1e-12)
    return jnp.where(nrm <= 1.0, pos, (2.0 - 1.0 / safe) * pos / safe) / 2.0


def _codes_body(o_ref, d_ref, t_ref, sidx_ref, out_ref):
    flat = _contracted(o_ref[...], d_ref[...], t_ref[...])
    iv3 = ((flat + 1.0) * GRID - 1.0) / 2.0
    vf3 = jnp.floor(iv3)
    vi3 = vf3.astype(jnp.int32) + 1
    fr3 = (iv3 > vf3).astype(jnp.int32)
    code = jnp.zeros((1, CB), jnp.int32)
    for axis in range(3):
        code = code + (vi3[axis:axis + 1, :] * jnp.int32(1 << (8 * axis))
                       + fr3[axis:axis + 1, :] * jnp.int32(1 << (24 + axis)))
    code = jnp.where(sidx_ref[...] < NS, code, jnp.int32(0x00FFFFFF))
    out_ref[...] = code


def _main_body(o_ref, d_ref, mask_ref, t_ref, dist_ref, sidx_ref, sel_ref,
               W1_ref, b1_ref, W2_ref, b2_ref, Ws_ref, bs_ref,
               Wr1a_ref, Wr1b_ref, br1_ref, Wr2_ref, br2_ref, out_ref):
    d3 = d_ref[...]
    flat = _contracted(o_ref[...], d3, t_ref[...])
    msk = mask_ref[...]
    h = jnp.maximum(jnp.dot(W1_ref[...], flat) + b1_ref[...], 0.0)
    feat = (jnp.dot(W2_ref[...], h) + b2_ref[...]) * msk
    sig = jnp.dot(Ws_ref[...], feat) + bs_ref[...]
    sigma = jax.nn.softplus(sig) * msk
    alpha = -sigma * dist_ref[...]
    sidx = sidx_ref[...]
    c = alpha
    for k in (1, 2, 4, 8, 16):
        c = c + jnp.where(sidx >= k, pltpu.roll(c, k, 1), 0.0)
    trans = jnp.exp(c - alpha)
    a = 1.0 - jnp.exp(alpha)
    w = trans * a
    wm2 = jnp.where((msk > 0.0) & (w > 1e-4), w, 0.0)
    hr = jnp.maximum(jnp.dot(Wr1a_ref[...], feat)
                     + jnp.dot(Wr1b_ref[...], d3) + br1_ref[...], 0.0)
    rgb = jax.nn.sigmoid(jnp.dot(Wr2_ref[...], hr) + br2_ref[...])
    contrib = rgb * wm2
    out_ref[...] = jnp.dot(contrib, sel_ref[...])


def _sc_mask_body(words_hbm, codes_hbm, out_hbm, bits_v, codes_v, mask_v):
    nc = 2
    wid = lax.axis_index("s") * nc + lax.axis_index("c")
    base = wid * PER_W
    pltpu.sync_copy(words_hbm, bits_v)
    pltpu.sync_copy(codes_hbm.at[pl.ds(base, PER_W)], codes_v)

    def body(i, carry):
        code = codes_v[pl.ds(i * 16, 16)]
        x0 = (code & 255) - 1
        y0 = ((code >> 8) & 255) - 1
        z0 = ((code >> 16) & 255) - 1
        fx = ((code >> 24) & 1) == 1
        fy = ((code >> 25) & 1) == 1
        fz = ((code >> 26) & 1) == 1
        lim = GRID - 1
        okx = ((x0 >= 0) & (x0 <= lim), fx & (x0 + 1 <= lim) & (x0 + 1 >= 0))
        oky = ((y0 >= 0) & (y0 <= lim), fy & (y0 + 1 <= lim) & (y0 + 1 >= 0))
        okz = ((z0 >= 0) & (z0 <= lim), fz & (z0 + 1 <= lim) & (z0 + 1 >= 0))
        m = None
        for dz in (0, 1):
            for dy in (0, 1):
                yz = (z0 + dz) * GRID + (y0 + dy)
                for dx in (0, 1):
                    x = x0 + dx
                    wi = (yz * 4 + (x >> 5)) & (NWORDS - 1)
                    wv = plsc.load_gather(bits_v, [wi])
                    hit = (((wv >> (x & 31)) & 1) == 1)
                    hit = hit & okz[dz] & oky[dy] & okx[dx]
                    m = hit if m is None else (m | hit)
        mask_v[pl.ds(i * 16, 16)] = jnp.where(m, 1.0, 0.0)
        return carry

    lax.fori_loop(0, VECS, body, 0)
    pltpu.sync_copy(mask_v, out_hbm.at[pl.ds(base, PER_W)])


@functools.cache
def _sc_mask_call():
    mesh = plsc.VectorSubcoreMesh(core_axis_name="c", subcore_axis_name="s")
    return pl.kernel(
        _sc_mask_body,
        out_type=jax.ShapeDtypeStruct((NROWS,), jnp.float32),
        mesh=mesh,
        compiler_params=pltpu.CompilerParams(needs_layout_passes=False),
        scratch_types=[
            pltpu.VMEM((NWORDS,), jnp.int32),
            pltpu.VMEM((PER_W,), jnp.int32),
            pltpu.VMEM((PER_W,), jnp.float32),
        ],
    )


@functools.cache
def _pack_call():
    return pl.pallas_call(
        _pack_body,
        grid=(NWORDS // PACK_ROWS,),
        in_specs=[pl.BlockSpec((PACK_ROWS, 32), lambda i: (i, 0))],
        out_specs=pl.BlockSpec((PACK_ROWS, 1), lambda i: (i, 0)),
        out_shape=jax.ShapeDtypeStruct((NWORDS, 1), jnp.int32),
    )


@functools.cache
def _codes_call():
    full = lambda i: (0, 0)
    return pl.pallas_call(
        _codes_body,
        grid=(NBLK,),
        in_specs=[
            pl.BlockSpec((3, CB), lambda i: (0, i)),
            pl.BlockSpec((3, CB), lambda i: (0, i)),
            pl.BlockSpec((1, CB), full),
            pl.BlockSpec((1, CB), full),
        ],
        out_specs=pl.BlockSpec((1, CB), lambda i: (0, i)),
        out_shape=jax.ShapeDtypeStruct((1, NROWS), jnp.int32),
    )


@functools.cache
def _main_call():
    full = lambda i: (0, 0)
    return pl.pallas_call(
        _main_body,
        grid=(NBLK,),
        in_specs=[
            pl.BlockSpec((3, CB), lambda i: (0, i)),
            pl.BlockSpec((3, CB), lambda i: (0, i)),
            pl.BlockSpec((1, CB), lambda i: (0, i)),
            pl.BlockSpec((1, CB), full),
            pl.BlockSpec((1, CB), full),
            pl.BlockSpec((1, CB), full),
            pl.BlockSpec((CB, RB), full),
            pl.BlockSpec((HID, 3), full),
            pl.BlockSpec((HID, 1), full),
            pl.BlockSpec((FEAT, HID), full),
            pl.BlockSpec((FEAT, 1), full),
            pl.BlockSpec((1, FEAT), full),
            pl.BlockSpec((1, 1), full),
            pl.BlockSpec((HID, FEAT), full),
            pl.BlockSpec((HID, 3), full),
            pl.BlockSpec((HID, 1), full),
            pl.BlockSpec((3, HID), full),
            pl.BlockSpec((3, 1), full),
        ],
        out_specs=pl.BlockSpec((3, RB), lambda i: (0, i)),
        out_shape=jax.ShapeDtypeStruct((3, NRAYS), jnp.float32),
    )


def kernel(rays_o, rays_d, grid, W1, b1, W2, b2, Ws, bs, Wr1, br1, Wr2, br2):
    trow = jnp.asarray(_T_ROW)
    drow = jnp.asarray(_D_ROW)
    srow = jnp.asarray(_S_ROW)
    sel = jnp.asarray(_SEL)
    oT = jnp.broadcast_to(rays_o.T[:, :, None],
                          (3, NRAYS, NSP)).reshape(3, NROWS)
    dT = jnp.broadcast_to(rays_d.T[:, :, None],
                          (3, NRAYS, NSP)).reshape(3, NROWS)
    words = _pack_call()(grid.reshape(NWORDS, 32)).reshape(NWORDS)
    codes = _codes_call()(oT, dT, trow, srow).reshape(NROWS)
    maskf = _sc_mask_call()(words, codes)
    outT = _main_call()(
        oT, dT, maskf.reshape(1, NROWS), trow, drow, srow, sel,
        W1.T, b1.reshape(HID, 1), W2.T, b2.reshape(FEAT, 1),
        Ws.T, bs.reshape(1, 1),
        Wr1[:FEAT].T, Wr1[FEAT:].T, br1.reshape(HID, 1),
        Wr2.T, br2.reshape(3, 1))
    return outT.T


# speedup vs baseline: 4.4438x; 1.1456x over previous
import functools
import math

import jax
import jax.numpy as jnp
import numpy as np
from jax import lax
from jax.experimental import pallas as pl
from jax.experimental.pallas import tpu as pltpu
from jax.experimental.pallas import tpu_sc as plsc

GRID = 128
FEAT = 64
HID = 64
NRAYS = 8192
DELTA_MIN = math.sqrt(3.0) / 1024.0


def _steps():
    t = 0.0
    ts = [t]
    ds = []
    while t < 100000.0:
        s = min(1e10, max(DELTA_MIN, t))
        t += s
        ts.append(t)
        ds.append(s)
    return np.asarray(ts[:-1], np.float32), np.asarray(ds, np.float32)


_T, _D = _steps()
NS = _T.shape[0]
NSP = 32
NROWS = NRAYS * NSP
RB = 256
CB = RB * NSP
NBLK = NRAYS // RB
NWORDS = GRID * GRID * (GRID // 32)
PACK_ROWS = 16384
NW = 32
PER_W = NROWS // NW
VECS = PER_W // 16

_T_PAD = np.concatenate([_T, np.zeros(NSP - NS, np.float32)])
_D_PAD = np.concatenate([_D, np.zeros(NSP - NS, np.float32)])
_T_ROW = np.tile(_T_PAD, RB)[None, :]
_D_ROW = np.tile(_D_PAD, RB)[None, :]
_S_ROW = np.tile(np.arange(NSP, dtype=np.int32), RB)[None, :]
_SEL = np.zeros((CB, RB), np.float32)
_SEL[np.arange(CB), np.arange(CB) // NSP] = 1.0


def _pack_body(g_ref, out_ref):
    b = (g_ref[...] != 0.0).astype(jnp.int32)
    lane = lax.broadcasted_iota(jnp.int32, b.shape, 1)
    s = b << (lane & 31)
    lm = lane & 31
    for k in (1, 2, 4, 8, 16):
        s = s + jnp.where(lm <= 31 - k, pltpu.roll(s, 128 - k, 1), 0)
    cols = [s[:, j * 32:j * 32 + 1] for j in range(4)]
    out_ref[...] = jnp.concatenate(cols, axis=1)


def _contracted(o3, d3, t):
    pos = o3 + d3 * t
    nrm = jnp.max(jnp.abs(pos), axis=0, keepdims=True)
    safe = jnp.maximum(nrm, 1e-12)
    return jnp.where(nrm <= 1.0, pos, (2.0 - 1.0 / safe) * pos / safe) / 2.0


def _codes_body(o_ref, d_ref, t_ref, sidx_ref, out_ref):
    flat = _contracted(o_ref[...], d_ref[...], t_ref[...])
    iv3 = ((flat + 1.0) * GRID - 1.0) / 2.0
    vf3 = jnp.floor(iv3)
    vi3 = vf3.astype(jnp.int32) + 1
    fr3 = (iv3 > vf3).astype(jnp.int32)
    code = jnp.zeros((1, CB), jnp.int32)
    for axis in range(3):
        code = code + (vi3[axis:axis + 1, :] * jnp.int32(1 << (8 * axis))
                       + fr3[axis:axis + 1, :] * jnp.int32(1 << (24 + axis)))
    code = jnp.where(sidx_ref[...] < NS, code, jnp.int32(0x00FFFFFF))
    out_ref[...] = code


def _main_body(o_ref, d_ref, mask_ref, t_ref, dist_ref, sidx_ref, sel_ref,
               W1_ref, b1_ref, W2_ref, b2_ref, Ws_ref, bs_ref,
               Wr1a_ref, Wr1b_ref, br1_ref, Wr2_ref, br2_ref, out_ref):
    d3 = d_ref[...]
    flat = _contracted(o_ref[...], d3, t_ref[...])
    msk = mask_ref[...]
    h = jnp.maximum(jnp.dot(W1_ref[...], flat) + b1_ref[...], 0.0)
    feat = (jnp.dot(W2_ref[...], h) + b2_ref[...]) * msk
    sig = jnp.dot(Ws_ref[...], feat) + bs_ref[...]
    sigma = jax.nn.softplus(sig) * msk
    alpha = -sigma * dist_ref[...]
    sidx = sidx_ref[...]
    c = alpha
    for k in (1, 2, 4, 8, 16):
        c = c + jnp.where(sidx >= k, pltpu.roll(c, k, 1), 0.0)
    trans = jnp.exp(c - alpha)
    a = 1.0 - jnp.exp(alpha)
    w = trans * a
    wm2 = jnp.where((msk > 0.0) & (w > 1e-4), w, 0.0)
    hr = jnp.maximum(jnp.dot(Wr1a_ref[...], feat)
                     + jnp.dot(Wr1b_ref[...], d3) + br1_ref[...], 0.0)
    rgb = jax.nn.sigmoid(jnp.dot(Wr2_ref[...], hr) + br2_ref[...])
    contrib = rgb * wm2
    out_ref[...] = jnp.dot(contrib, sel_ref[...])


def _sc_mask_body(words_hbm, codes_hbm, out_hbm, bits_v, codes_v, mask_v):
    nc = 2
    wid = lax.axis_index("s") * nc + lax.axis_index("c")
    base = wid * PER_W
    pltpu.sync_copy(words_hbm, bits_v)
    pltpu.sync_copy(codes_hbm.at[pl.ds(base, PER_W)], codes_v)

    def body(i, carry):
        code = codes_v[pl.ds(i * 16, 16)]
        x0 = (code & 255) - 1
        y0 = ((code >> 8) & 255) - 1
        z0 = ((code >> 16) & 255) - 1
        fx = ((code >> 24) & 1) == 1
        fy = ((code >> 25) & 1) == 1
        fz = ((code >> 26) & 1) == 1
        lim = GRID - 1
        okx = ((x0 >= 0) & (x0 <= lim), fx & (x0 + 1 <= lim) & (x0 + 1 >= 0))
        oky = ((y0 >= 0) & (y0 <= lim), fy & (y0 + 1 <= lim) & (y0 + 1 >= 0))
        okz = ((z0 >= 0) & (z0 <= lim), fz & (z0 + 1 <= lim) & (z0 + 1 >= 0))
        m = None
        for dz in (0, 1):
            for dy in (0, 1):
                yz = (z0 + dz) * GRID + (y0 + dy)
                for dx in (0, 1):
                    x = x0 + dx
                    wi = (yz * 4 + (x >> 5)) & (NWORDS - 1)
                    wv = plsc.load_gather(bits_v, [wi])
                    hit = (((wv >> (x & 31)) & 1) == 1)
                    hit = hit & okz[dz] & oky[dy] & okx[dx]
                    m = hit if m is None else (m | hit)
        mask_v[pl.ds(i * 16, 16)] = jnp.where(m, 1.0, 0.0)
        return carry

    lax.fori_loop(0, VECS, body, 0)
    pltpu.sync_copy(mask_v, out_hbm.at[pl.ds(base, PER_W)])


@functools.cache
def _sc_mask_call():
    mesh = plsc.VectorSubcoreMesh(core_axis_name="c", subcore_axis_name="s")
    return pl.kernel(
        _sc_mask_body,
        out_type=jax.ShapeDtypeStruct((NROWS,), jnp.float32),
        mesh=mesh,
        compiler_params=pltpu.CompilerParams(needs_layout_passes=False),
        scratch_types=[
            pltpu.VMEM((NWORDS,), jnp.int32),
            pltpu.VMEM((PER_W,), jnp.int32),
            pltpu.VMEM((PER_W,), jnp.float32),
        ],
    )


@functools.cache
def _pack_call():
    return pl.pallas_call(
        _pack_body,
        grid=(1,),
        in_specs=[pl.BlockSpec((PACK_ROWS, 128), lambda i: (0, 0))],
        out_specs=pl.BlockSpec((PACK_ROWS, 4), lambda i: (0, 0)),
        out_shape=jax.ShapeDtypeStruct((PACK_ROWS, 4), jnp.int32),
    )


@functools.cache
def _codes_call():
    full = lambda i: (0, 0)
    return pl.pallas_call(
        _codes_body,
        grid=(NBLK,),
        in_specs=[
            pl.BlockSpec((3, CB), lambda i: (0, i)),
            pl.BlockSpec((3, CB), lambda i: (0, i)),
            pl.BlockSpec((1, CB), full),
            pl.BlockSpec((1, CB), full),
        ],
        out_specs=pl.BlockSpec((1, CB), lambda i: (0, i)),
        out_shape=jax.ShapeDtypeStruct((1, NROWS), jnp.int32),
    )


@functools.cache
def _main_call():
    full = lambda i: (0, 0)
    return pl.pallas_call(
        _main_body,
        grid=(NBLK,),
        in_specs=[
            pl.BlockSpec((3, CB), lambda i: (0, i)),
            pl.BlockSpec((3, CB), lambda i: (0, i)),
            pl.BlockSpec((1, CB), lambda i: (0, i)),
            pl.BlockSpec((1, CB), full),
            pl.BlockSpec((1, CB), full),
            pl.BlockSpec((1, CB), full),
            pl.BlockSpec((CB, RB), full),
            pl.BlockSpec((HID, 3), full),
            pl.BlockSpec((HID, 1), full),
            pl.BlockSpec((FEAT, HID), full),
            pl.BlockSpec((FEAT, 1), full),
            pl.BlockSpec((1, FEAT), full),
            pl.BlockSpec((1, 1), full),
            pl.BlockSpec((HID, FEAT), full),
            pl.BlockSpec((HID, 3), full),
            pl.BlockSpec((HID, 1), full),
            pl.BlockSpec((3, HID), full),
            pl.BlockSpec((3, 1), full),
        ],
        out_specs=pl.BlockSpec((3, RB), lambda i: (0, i)),
        out_shape=jax.ShapeDtypeStruct((3, NRAYS), jnp.float32),
    )


def kernel(rays_o, rays_d, grid, W1, b1, W2, b2, Ws, bs, Wr1, br1, Wr2, br2):
    trow = jnp.asarray(_T_ROW)
    drow = jnp.asarray(_D_ROW)
    srow = jnp.asarray(_S_ROW)
    sel = jnp.asarray(_SEL)
    oT = jnp.broadcast_to(rays_o.T[:, :, None],
                          (3, NRAYS, NSP)).reshape(3, NROWS)
    dT = jnp.broadcast_to(rays_d.T[:, :, None],
                          (3, NRAYS, NSP)).reshape(3, NROWS)
    words = _pack_call()(grid.reshape(PACK_ROWS, 128)).reshape(NWORDS)
    codes = _codes_call()(oT, dT, trow, srow).reshape(NROWS)
    maskf = _sc_mask_call()(words, codes)
    outT = _main_call()(
        oT, dT, maskf.reshape(1, NROWS), trow, drow, srow, sel,
        W1.T, b1.reshape(HID, 1), W2.T, b2.reshape(FEAT, 1),
        Ws.T, bs.reshape(1, 1),
        Wr1[:FEAT].T, Wr1[FEAT:].T, br1.reshape(HID, 1),
        Wr2.T, br2.reshape(3, 1))
    return outT.T
